# Initial kernel scaffold; baseline (speedup 1.0000x reference)
import jax
import jax.numpy as jnp
from jax.experimental import pallas as pl


def kernel(x, edge_index, edge_attr, enc_node_W, enc_node_b, enc_edge_W, enc_edge_b, gat_W, gat_a_src, gat_a_dst, gat_b, dec_W1, dec_b1, dec_W2, dec_b2):
    raise NotImplementedError("write your pallas kernel here")


# speedup vs baseline: 9.7841x; 9.7841x over previous
import functools

import jax
import jax.numpy as jnp
from jax import lax
from jax.experimental import pallas as pl
from jax.experimental.pallas import tpu as pltpu
from jax.experimental.pallas import tpu_sc as plsc

NC = 2
NS = 16
NW = NC * NS


def _mm_bias_relu_body(x_ref, w_ref, b_ref, o_ref):
    y = jnp.dot(x_ref[...], w_ref[...], preferred_element_type=jnp.float32)
    o_ref[...] = jnp.maximum(y + b_ref[...], 0.0)


def _mm_bias_relu(x, w, b2d, bm):
    m, k = x.shape
    n = w.shape[1]
    return pl.pallas_call(
        _mm_bias_relu_body,
        grid=(m // bm,),
        in_specs=[
            pl.BlockSpec((bm, k), lambda i: (i, 0)),
            pl.BlockSpec((k, n), lambda i: (0, 0)),
            pl.BlockSpec((1, n), lambda i: (0, 0)),
        ],
        out_specs=pl.BlockSpec((bm, n), lambda i: (i, 0)),
        out_shape=jax.ShapeDtypeStruct((m, n), jnp.float32),
    )(x, w, b2d)


def _gat_dense_body(h_ref, w_ref, a_ref, hw_ref, ss_ref, sd_ref):
    hw = jnp.dot(h_ref[...], w_ref[...], preferred_element_type=jnp.float32)
    hw_ref[...] = hw
    sv = jnp.dot(hw, a_ref[...], preferred_element_type=jnp.float32)
    ss_ref[...] = sv[:, 0:1]
    sd_ref[...] = sv[:, 1:2]


def _gat_dense(h, w, a2, bm):
    m, k = h.shape
    n = w.shape[1]
    return pl.pallas_call(
        _gat_dense_body,
        grid=(m // bm,),
        in_specs=[
            pl.BlockSpec((bm, k), lambda i: (i, 0)),
            pl.BlockSpec((k, n), lambda i: (0, 0)),
            pl.BlockSpec((n, 2), lambda i: (0, 0)),
        ],
        out_specs=[
            pl.BlockSpec((bm, n), lambda i: (i, 0)),
            pl.BlockSpec((bm, 1), lambda i: (i, 0)),
            pl.BlockSpec((bm, 1), lambda i: (i, 0)),
        ],
        out_shape=[
            jax.ShapeDtypeStruct((m, n), jnp.float32),
            jax.ShapeDtypeStruct((m, 1), jnp.float32),
            jax.ShapeDtypeStruct((m, 1), jnp.float32),
        ],
    )(h, w, a2)


def _combine_body(pn_ref, pd_ref, b_ref, o_ref):
    num = pn_ref[0] + pn_ref[1]
    den = pd_ref[0] + pd_ref[1]
    o_ref[...] = jnp.maximum(num / (den + 1e-16) + b_ref[...], 0.0)


def _combine(pn, dcol, b2d, bm):
    _, m, n = pn.shape
    return pl.pallas_call(
        _combine_body,
        grid=(m // bm,),
        in_specs=[
            pl.BlockSpec((2, bm, n), lambda i: (0, i, 0)),
            pl.BlockSpec((2, bm, 1), lambda i: (0, i, 0)),
            pl.BlockSpec((1, n), lambda i: (0, 0)),
        ],
        out_specs=pl.BlockSpec((bm, n), lambda i: (i, 0)),
        out_shape=jax.ShapeDtypeStruct((m, n), jnp.float32),
    )(pn, dcol, b2d)


def _dec_tables_body(h_ref, wa_ref, wb_ref, a_ref, b_ref):
    a_ref[...] = jnp.dot(h_ref[...], wa_ref[...], preferred_element_type=jnp.float32)
    b_ref[...] = jnp.dot(h_ref[...], wb_ref[...], preferred_element_type=jnp.float32)


def _dec_tables(h, wa, wb, bm):
    m, k = h.shape
    n = wa.shape[1]
    return pl.pallas_call(
        _dec_tables_body,
        grid=(m // bm,),
        in_specs=[
            pl.BlockSpec((bm, k), lambda i: (i, 0)),
            pl.BlockSpec((k, n), lambda i: (0, 0)),
            pl.BlockSpec((k, n), lambda i: (0, 0)),
        ],
        out_specs=[
            pl.BlockSpec((bm, n), lambda i: (i, 0)),
            pl.BlockSpec((bm, n), lambda i: (i, 0)),
        ],
        out_shape=[
            jax.ShapeDtypeStruct((m, n), jnp.float32),
            jax.ShapeDtypeStruct((m, n), jnp.float32),
        ],
    )(h, wa, wb)


def _edge_mlp_body(ea_ref, we_ref, be_ref, w1c_ref, b1_ref, o_ref):
    he = jnp.dot(ea_ref[...], we_ref[...], preferred_element_type=jnp.float32)
    he = jnp.maximum(he + be_ref[...], 0.0)
    o_ref[...] = jnp.dot(he, w1c_ref[...], preferred_element_type=jnp.float32) + b1_ref[...]


def _edge_mlp(ea, we, be2d, w1c, b12d, bm):
    m, k = ea.shape
    n = we.shape[1]
    return pl.pallas_call(
        _edge_mlp_body,
        grid=(m // bm,),
        in_specs=[
            pl.BlockSpec((bm, k), lambda i: (i, 0)),
            pl.BlockSpec((k, n), lambda i: (0, 0)),
            pl.BlockSpec((1, n), lambda i: (0, 0)),
            pl.BlockSpec((n, n), lambda i: (0, 0)),
            pl.BlockSpec((1, n), lambda i: (0, 0)),
        ],
        out_specs=pl.BlockSpec((bm, n), lambda i: (i, 0)),
        out_shape=jax.ShapeDtypeStruct((m, n), jnp.float32),
    )(ea, we, be2d, w1c, b12d)


def _dec_reduce_body(p_ref, b2_ref, o_ref):
    o_ref[...] = jnp.sum(p_ref[...], axis=1, keepdims=True) + b2_ref[...]


def _dec_reduce(p, b2d, bm):
    m = p.shape[0]
    return pl.pallas_call(
        _dec_reduce_body,
        grid=(m // bm,),
        in_specs=[
            pl.BlockSpec((bm, 16), lambda i: (i, 0)),
            pl.BlockSpec((1, 1), lambda i: (0, 0)),
        ],
        out_specs=pl.BlockSpec((bm, 1), lambda i: (i, 0)),
        out_shape=jax.ShapeDtypeStruct((m, 1), jnp.float32),
    )(p, b2d)


def _gat_edge_sc(src, dst, ss, sd, hw, *, n, e, h, c):
    ept = e // NW
    nchunk = ept // c
    kh = h // 16
    nrc = (n // 16 + NS - 1) // NS
    nd = ((n // 128) + NS - 1) // NS * NS
    ndc = (nd // 16 + NS - 1) // NS
    maxcd = nd // 16 - 1

    mesh = plsc.VectorSubcoreMesh(core_axis_name="c", subcore_axis_name="s",
                                  num_cores=NC, num_subcores=NS)

    @functools.partial(
        pl.kernel,
        out_type=(
            jax.ShapeDtypeStruct((NC * n, h), jnp.float32),
            jax.ShapeDtypeStruct((NC * nd, 128), jnp.float32),
        ),
        mesh=mesh,
        compiler_params=pltpu.CompilerParams(needs_layout_passes=False),
        scratch_types=[
            pltpu.VMEM((c,), jnp.int32),
            pltpu.VMEM((c,), jnp.int32),
            pltpu.VMEM((c,), jnp.int32),
            pltpu.VMEM((c, h), jnp.float32),
            pltpu.VMEM((c,), jnp.float32),
            pltpu.VMEM((c,), jnp.float32),
            pltpu.VMEM((c,), jnp.float32),
            pltpu.VMEM((c, 128), jnp.float32),
            pltpu.VMEM((16,), jnp.int32),
            pltpu.VMEM((16, h), jnp.float32),
            pltpu.VMEM((16, 128), jnp.float32),
            pltpu.VMEM_SHARED((n, h), jnp.float32),
            pltpu.VMEM_SHARED((nd, 128), jnp.float32),
            pltpu.SemaphoreType.DMA,
        ],
    )
    def k(src_hbm, dst_hbm, ss_hbm, sd_hbm, hw_hbm, pn_hbm, pd_hbm,
          srcv, dstv, d8v, rows, ssg, sdg, wv, wst, ridx, bn, bd,
          accn, accd, sem):
        cid = lax.axis_index("c")
        sid = lax.axis_index("s")
        wid = cid * NS + sid

        lane = lax.iota(jnp.int32, 16)
        zv = jnp.zeros((16,), jnp.float32)
        for r in range(16):
            for kk in range(kh):
                bn[r, pl.ds(kk * 16, 16)] = zv
        for r in range(c):
            for kk in range(8):
                wst[r, pl.ds(kk * 16, 16)] = zv

        maxc = n // 16 - 1

        def zchunk(g, _):
            ch = jnp.minimum(g * NS + sid, maxc)
            ridx[:] = ch * 16 + lane
            pltpu.sync_copy(bn, accn.at[ridx])
            return 0

        lax.fori_loop(0, nrc, zchunk, 0)

        def zchunkd(g, _):
            ch = jnp.minimum(g * NS + sid, maxcd)
            ridx[:] = ch * 16 + lane
            pltpu.sync_copy(bn, accd.at[ridx])
            return 0

        lax.fori_loop(0, ndc, zchunkd, 0)
        plsc.subcore_barrier()

        ebase = wid * ept

        def chunk_body(cc, _):
            off = ebase + cc * c
            pltpu.sync_copy(src_hbm.at[pl.ds(off, c)], srcv)
            pltpu.sync_copy(dst_hbm.at[pl.ds(off, c)], dstv)
            cp1 = pltpu.async_copy(hw_hbm.at[srcv], rows, sem)
            cp2 = pltpu.async_copy(ss_hbm.at[srcv], ssg, sem)
            cp3 = pltpu.async_copy(sd_hbm.at[dstv], sdg, sem)
            cp1.wait()
            cp2.wait()
            cp3.wait()
            for g in range(c // 16):
                s = ssg[pl.ds(g * 16, 16)] + sdg[pl.ds(g * 16, 16)]
                s = jnp.where(s > 0, s, 0.2 * s)
                w = jnp.exp(s)
                wv[pl.ds(g * 16, 16)] = w
                di = dstv[pl.ds(g * 16, 16)]
                d8v[pl.ds(g * 16, 16)] = lax.shift_right_logical(di, 7)
                plsc.store_scatter(wst, [g * 16 + lane, di & 127], w)

            def edge_body(ee, _):
                wsp = plsc.load_gather(wv, [jnp.full((16,), ee, jnp.int32)])
                for kk in range(kh):
                    rows[ee, pl.ds(kk * 16, 16)] = rows[ee, pl.ds(kk * 16, 16)] * wsp
                return 0

            lax.fori_loop(0, c, edge_body, 0)
            pltpu.sync_copy(rows, accn.at[dstv], add=True)
            pltpu.sync_copy(wst, accd.at[d8v], add=True)
            for g in range(c // 16):
                di = dstv[pl.ds(g * 16, 16)]
                plsc.store_scatter(wst, [g * 16 + lane, di & 127], zv)
            return 0

        lax.fori_loop(0, nchunk, chunk_body, 0)
        plsc.subcore_barrier()

        def cchunk(g, _):
            ch = jnp.minimum(g * NS + sid, maxc)
            ridx[:] = ch * 16 + lane
            pltpu.async_copy(accn.at[ridx], bn, sem).wait()
            r = pl.multiple_of(cid * n, 8) + pl.multiple_of(ch * 16, 8)
            pltpu.sync_copy(bn, pn_hbm.at[pl.ds(r, 16)])
            return 0

        lax.fori_loop(0, nrc, cchunk, 0)

        def cchunkd(g, _):
            ch = jnp.minimum(g * NS + sid, maxcd)
            ridx[:] = ch * 16 + lane
            pltpu.async_copy(accd.at[ridx], bd, sem).wait()
            r = pl.multiple_of(cid * nd, 8) + pl.multiple_of(ch * 16, 8)
            pltpu.sync_copy(bd, pd_hbm.at[pl.ds(r, 16)])
            return 0

        lax.fori_loop(0, ndc, cchunkd, 0)

    pn2, pd2 = k(src, dst, ss, sd, hw)
    return pn2.reshape(NC, n, h), pd2.reshape(NC, nd, 128)


def _dec_edge_sc(src, dst, ha, hb, hew, w2, *, n, e, h, c):
    ept = e // NW
    nchunk = ept // c
    kh = h // 16

    mesh = plsc.VectorSubcoreMesh(core_axis_name="c", subcore_axis_name="s", num_cores=NC, num_subcores=NS)

    @functools.partial(
        pl.kernel,
        out_type=jax.ShapeDtypeStruct((e, 16), jnp.float32),
        mesh=mesh,
        compiler_params=pltpu.CompilerParams(needs_layout_passes=False),
        scratch_types=[
            pltpu.VMEM((c,), jnp.int32),
            pltpu.VMEM((c,), jnp.int32),
            pltpu.VMEM((c, h), jnp.float32),
            pltpu.VMEM((c, h), jnp.float32),
            pltpu.VMEM((c, h), jnp.float32),
            pltpu.VMEM((h,), jnp.float32),
            pltpu.VMEM((c, 16), jnp.float32),
            pltpu.SemaphoreType.DMA,
        ],
    )
    def k(src_hbm, dst_hbm, ha_hbm, hb_hbm, hew_hbm, w2_hbm, out_hbm, srcv,
          dstv, rowsa, rowsb, rowse, w2v, outv, sem):
        cid = lax.axis_index("c")
        sid = lax.axis_index("s")
        wid = cid * NS + sid
        pltpu.sync_copy(w2_hbm, w2v)
        ebase = wid * ept

        def chunk_body(cc, _):
            off = ebase + cc * c
            pltpu.sync_copy(src_hbm.at[pl.ds(off, c)], srcv)
            pltpu.sync_copy(dst_hbm.at[pl.ds(off, c)], dstv)
            pltpu.async_copy(ha_hbm.at[srcv], rowsa, sem).wait()
            pltpu.async_copy(hb_hbm.at[dstv], rowsb, sem).wait()
            pltpu.sync_copy(hew_hbm.at[pl.ds(off, c)], rowse)

            def edge_body(ee, _):
                acc = jnp.zeros((16,), jnp.float32)
                for kk in range(kh):
                    q = (rowsa[ee, pl.ds(kk * 16, 16)]
                         + rowsb[ee, pl.ds(kk * 16, 16)]
                         + rowse[ee, pl.ds(kk * 16, 16)])
                    q = jnp.maximum(q, 0.0)
                    acc = acc + q * w2v[pl.ds(kk * 16, 16)]
                outv[ee, :] = acc
                return 0

            lax.fori_loop(0, c, edge_body, 0)
            pltpu.sync_copy(outv, out_hbm.at[pl.ds(off, c)])
            return 0

        lax.fori_loop(0, nchunk, chunk_body, 0)

    return k(src, dst, ha, hb, hew, w2)


def kernel(x, edge_index, edge_attr, enc_node_W, enc_node_b, enc_edge_W,
           enc_edge_b, gat_W, gat_a_src, gat_a_dst, gat_b, dec_W1, dec_b1,
           dec_W2, dec_b2):
    n, dn = x.shape
    e = edge_index.shape[1]
    hh = enc_node_W.shape[1]
    ll = gat_W.shape[0]

    bm_n = 1000 if n % 1000 == 0 else n
    bm_e = 2000 if e % 2000 == 0 else e
    c = 80 if (e // NW) % 80 == 0 else 16

    src = edge_index[0]
    dst = edge_index[1]

    h = _mm_bias_relu(x, enc_node_W, enc_node_b.reshape(1, hh), bm_n)

    for l in range(ll):
        a2 = jnp.stack([gat_a_src[l], gat_a_dst[l]], axis=1)
        hw, ssc, sdc = _gat_dense(h, gat_W[l], a2, bm_n)
        pn, pd = _gat_edge_sc(src, dst, ssc[:, 0], sdc[:, 0], hw,
                              n=n, e=e, h=hh, c=c)
        dcol = pd.reshape(NC, -1, 1)[:, :n, :]
        h = _combine(pn, dcol, gat_b[l].reshape(1, hh), bm_n)

    w1a = dec_W1[:hh]
    w1b = dec_W1[hh:2 * hh]
    w1c = dec_W1[2 * hh:]
    ha, hb = _dec_tables(h, w1a, w1b, bm_n)
    hew = _edge_mlp(edge_attr, enc_edge_W, enc_edge_b.reshape(1, hh), w1c,
                    dec_b1.reshape(1, hh), bm_e)
    part = _dec_edge_sc(src, dst, ha, hb, hew, dec_W2[:, 0],
                        n=n, e=e, h=hh, c=c)
    out = _dec_reduce(part, dec_b2.reshape(1, 1), bm_e)
    return out


# speedup vs baseline: 10.4063x; 1.0636x over previous
import functools

import jax
import jax.numpy as jnp
from jax import lax
from jax.experimental import pallas as pl
from jax.experimental.pallas import tpu as pltpu
from jax.experimental.pallas import tpu_sc as plsc

NC = 2
NS = 16
NW = NC * NS


def _mm_bias_relu_body(x_ref, w_ref, b_ref, o_ref):
    y = jnp.dot(x_ref[...], w_ref[...], preferred_element_type=jnp.float32)
    o_ref[...] = jnp.maximum(y + b_ref[...], 0.0)


def _mm_bias_relu(x, w, b2d, bm):
    m, k = x.shape
    n = w.shape[1]
    return pl.pallas_call(
        _mm_bias_relu_body,
        grid=(m // bm,),
        in_specs=[
            pl.BlockSpec((bm, k), lambda i: (i, 0)),
            pl.BlockSpec((k, n), lambda i: (0, 0)),
            pl.BlockSpec((1, n), lambda i: (0, 0)),
        ],
        out_specs=pl.BlockSpec((bm, n), lambda i: (i, 0)),
        out_shape=jax.ShapeDtypeStruct((m, n), jnp.float32),
    )(x, w, b2d)


def _gat_dense_body(h_ref, w_ref, a_ref, hw_ref, ss_ref, sd_ref):
    hw = jnp.dot(h_ref[...], w_ref[...], preferred_element_type=jnp.float32)
    hw_ref[...] = hw
    sv = jnp.dot(hw, a_ref[...], preferred_element_type=jnp.float32)
    ss_ref[...] = sv[:, 0:1]
    sd_ref[...] = sv[:, 1:2]


def _gat_dense(h, w, a2, bm):
    m, k = h.shape
    n = w.shape[1]
    return pl.pallas_call(
        _gat_dense_body,
        grid=(m // bm,),
        in_specs=[
            pl.BlockSpec((bm, k), lambda i: (i, 0)),
            pl.BlockSpec((k, n), lambda i: (0, 0)),
            pl.BlockSpec((n, 2), lambda i: (0, 0)),
        ],
        out_specs=[
            pl.BlockSpec((bm, n), lambda i: (i, 0)),
            pl.BlockSpec((bm, 1), lambda i: (i, 0)),
            pl.BlockSpec((bm, 1), lambda i: (i, 0)),
        ],
        out_shape=[
            jax.ShapeDtypeStruct((m, n), jnp.float32),
            jax.ShapeDtypeStruct((m, 1), jnp.float32),
            jax.ShapeDtypeStruct((m, 1), jnp.float32),
        ],
    )(h, w, a2)


def _combine_body(pn_ref, pd_ref, b_ref, o_ref):
    num = pn_ref[0] + pn_ref[1]
    den = pd_ref[0] + pd_ref[1]
    o_ref[...] = jnp.maximum(num / (den + 1e-16) + b_ref[...], 0.0)


def _combine(pn, dcol, b2d, bm):
    _, m, n = pn.shape
    return pl.pallas_call(
        _combine_body,
        grid=(m // bm,),
        in_specs=[
            pl.BlockSpec((2, bm, n), lambda i: (0, i, 0)),
            pl.BlockSpec((2, bm, 1), lambda i: (0, i, 0)),
            pl.BlockSpec((1, n), lambda i: (0, 0)),
        ],
        out_specs=pl.BlockSpec((bm, n), lambda i: (i, 0)),
        out_shape=jax.ShapeDtypeStruct((m, n), jnp.float32),
    )(pn, dcol, b2d)


def _dec_tables_body(h_ref, wa_ref, wb_ref, a_ref, b_ref):
    a_ref[...] = jnp.dot(h_ref[...], wa_ref[...], preferred_element_type=jnp.float32)
    b_ref[...] = jnp.dot(h_ref[...], wb_ref[...], preferred_element_type=jnp.float32)


def _dec_tables(h, wa, wb, bm):
    m, k = h.shape
    n = wa.shape[1]
    return pl.pallas_call(
        _dec_tables_body,
        grid=(m // bm,),
        in_specs=[
            pl.BlockSpec((bm, k), lambda i: (i, 0)),
            pl.BlockSpec((k, n), lambda i: (0, 0)),
            pl.BlockSpec((k, n), lambda i: (0, 0)),
        ],
        out_specs=[
            pl.BlockSpec((bm, n), lambda i: (i, 0)),
            pl.BlockSpec((bm, n), lambda i: (i, 0)),
        ],
        out_shape=[
            jax.ShapeDtypeStruct((m, n), jnp.float32),
            jax.ShapeDtypeStruct((m, n), jnp.float32),
        ],
    )(h, wa, wb)


def _edge_mlp_body(ea_ref, we_ref, be_ref, w1c_ref, b1_ref, o_ref):
    he = jnp.dot(ea_ref[...], we_ref[...], preferred_element_type=jnp.float32)
    he = jnp.maximum(he + be_ref[...], 0.0)
    o_ref[...] = jnp.dot(he, w1c_ref[...], preferred_element_type=jnp.float32) + b1_ref[...]


def _edge_mlp(ea, we, be2d, w1c, b12d, bm):
    m, k = ea.shape
    n = we.shape[1]
    return pl.pallas_call(
        _edge_mlp_body,
        grid=(m // bm,),
        in_specs=[
            pl.BlockSpec((bm, k), lambda i: (i, 0)),
            pl.BlockSpec((k, n), lambda i: (0, 0)),
            pl.BlockSpec((1, n), lambda i: (0, 0)),
            pl.BlockSpec((n, n), lambda i: (0, 0)),
            pl.BlockSpec((1, n), lambda i: (0, 0)),
        ],
        out_specs=pl.BlockSpec((bm, n), lambda i: (i, 0)),
        out_shape=jax.ShapeDtypeStruct((m, n), jnp.float32),
    )(ea, we, be2d, w1c, b12d)


def _dec_reduce_body(p_ref, b2_ref, o_ref):
    o_ref[...] = jnp.sum(p_ref[...], axis=1, keepdims=True) + b2_ref[...]


def _dec_reduce(p, b2d, bm):
    m = p.shape[0]
    return pl.pallas_call(
        _dec_reduce_body,
        grid=(m // bm,),
        in_specs=[
            pl.BlockSpec((bm, 16), lambda i: (i, 0)),
            pl.BlockSpec((1, 1), lambda i: (0, 0)),
        ],
        out_specs=pl.BlockSpec((bm, 1), lambda i: (i, 0)),
        out_shape=jax.ShapeDtypeStruct((m, 1), jnp.float32),
    )(p, b2d)


def _gat_edge_sc(src, dst, ss, sd, hw, *, n, e, h, c):
    ept = e // NW
    nchunk = ept // c
    kh = h // 16
    nrc = (n // 16 + NS - 1) // NS
    nd = ((n // 128) + NS - 1) // NS * NS
    ndc = (nd // 16 + NS - 1) // NS
    maxcd = nd // 16 - 1

    mesh = plsc.VectorSubcoreMesh(core_axis_name="c", subcore_axis_name="s",
                                  num_cores=NC, num_subcores=NS)

    @functools.partial(
        pl.kernel,
        out_type=(
            jax.ShapeDtypeStruct((NC * n, h), jnp.float32),
            jax.ShapeDtypeStruct((NC * nd, 128), jnp.float32),
        ),
        mesh=mesh,
        compiler_params=pltpu.CompilerParams(needs_layout_passes=False),
        scratch_types=[
            pltpu.VMEM((c,), jnp.int32),
            pltpu.VMEM((c,), jnp.int32),
            pltpu.VMEM((c,), jnp.int32),
            pltpu.VMEM((c, h), jnp.float32),
            pltpu.VMEM((c,), jnp.float32),
            pltpu.VMEM((c,), jnp.float32),
            pltpu.VMEM((c,), jnp.float32),
            pltpu.VMEM((c, 128), jnp.float32),
            pltpu.VMEM((16,), jnp.int32),
            pltpu.VMEM((16, h), jnp.float32),
            pltpu.VMEM((16, 128), jnp.float32),
            pltpu.VMEM_SHARED((n, h), jnp.float32),
            pltpu.VMEM_SHARED((nd, 128), jnp.float32),
            pltpu.SemaphoreType.DMA,
        ],
    )
    def k(src_hbm, dst_hbm, ss_hbm, sd_hbm, hw_hbm, pn_hbm, pd_hbm,
          srcv, dstv, d8v, rows, ssg, sdg, wv, wst, ridx, bn, bd,
          accn, accd, sem):
        cid = lax.axis_index("c")
        sid = lax.axis_index("s")
        wid = cid * NS + sid

        lane = lax.iota(jnp.int32, 16)
        zv = jnp.zeros((16,), jnp.float32)
        for r in range(16):
            for kk in range(kh):
                bn[r, pl.ds(kk * 16, 16)] = zv
        for r in range(c):
            for kk in range(8):
                wst[r, pl.ds(kk * 16, 16)] = zv

        maxc = n // 16 - 1

        def zchunk(g, _):
            ch = jnp.minimum(g * NS + sid, maxc)
            ridx[:] = ch * 16 + lane
            pltpu.sync_copy(bn, accn.at[ridx])
            return 0

        lax.fori_loop(0, nrc, zchunk, 0)

        def zchunkd(g, _):
            ch = jnp.minimum(g * NS + sid, maxcd)
            ridx[:] = ch * 16 + lane
            pltpu.sync_copy(bn, accd.at[ridx])
            return 0

        lax.fori_loop(0, ndc, zchunkd, 0)
        plsc.subcore_barrier()

        ebase = wid * ept

        def chunk_body(cc, _):
            off = ebase + cc * c
            pltpu.sync_copy(src_hbm.at[pl.ds(off, c)], srcv)
            pltpu.sync_copy(dst_hbm.at[pl.ds(off, c)], dstv)
            cp1 = pltpu.async_copy(hw_hbm.at[srcv], rows, sem)
            cp2 = pltpu.async_copy(ss_hbm.at[srcv], ssg, sem)
            cp3 = pltpu.async_copy(sd_hbm.at[dstv], sdg, sem)
            cp1.wait()
            cp2.wait()
            cp3.wait()
            for g in range(c // 16):
                s = ssg[pl.ds(g * 16, 16)] + sdg[pl.ds(g * 16, 16)]
                s = jnp.where(s > 0, s, 0.2 * s)
                w = jnp.exp(s)
                wv[pl.ds(g * 16, 16)] = w
                di = dstv[pl.ds(g * 16, 16)]
                d8v[pl.ds(g * 16, 16)] = lax.shift_right_logical(di, 7)
                plsc.store_scatter(wst, [g * 16 + lane, di & 127], w)

            @plsc.parallel_loop(0, c, unroll=4)
            def _(ee):
                wsp = plsc.load_gather(wv, [jnp.full((16,), ee, jnp.int32)])
                for kk in range(kh):
                    rows[ee, pl.ds(kk * 16, 16)] = rows[ee, pl.ds(kk * 16, 16)] * wsp
            pltpu.sync_copy(rows, accn.at[dstv], add=True)
            pltpu.sync_copy(wst, accd.at[d8v], add=True)
            for g in range(c // 16):
                di = dstv[pl.ds(g * 16, 16)]
                plsc.store_scatter(wst, [g * 16 + lane, di & 127], zv)
            return 0

        lax.fori_loop(0, nchunk, chunk_body, 0)
        plsc.subcore_barrier()

        def cchunk(g, _):
            ch = jnp.minimum(g * NS + sid, maxc)
            ridx[:] = ch * 16 + lane
            pltpu.async_copy(accn.at[ridx], bn, sem).wait()
            r = pl.multiple_of(cid * n, 8) + pl.multiple_of(ch * 16, 8)
            pltpu.sync_copy(bn, pn_hbm.at[pl.ds(r, 16)])
            return 0

        lax.fori_loop(0, nrc, cchunk, 0)

        def cchunkd(g, _):
            ch = jnp.minimum(g * NS + sid, maxcd)
            ridx[:] = ch * 16 + lane
            pltpu.async_copy(accd.at[ridx], bd, sem).wait()
            r = pl.multiple_of(cid * nd, 8) + pl.multiple_of(ch * 16, 8)
            pltpu.sync_copy(bd, pd_hbm.at[pl.ds(r, 16)])
            return 0

        lax.fori_loop(0, ndc, cchunkd, 0)

    pn2, pd2 = k(src, dst, ss, sd, hw)
    return pn2.reshape(NC, n, h), pd2.reshape(NC, nd, 128)


def _dec_edge_sc(src, dst, ha, hb, hew, w2, *, n, e, h, c):
    ept = e // NW
    nchunk = ept // c
    kh = h // 16

    mesh = plsc.VectorSubcoreMesh(core_axis_name="c", subcore_axis_name="s", num_cores=NC, num_subcores=NS)

    @functools.partial(
        pl.kernel,
        out_type=jax.ShapeDtypeStruct((e, 16), jnp.float32),
        mesh=mesh,
        compiler_params=pltpu.CompilerParams(needs_layout_passes=False),
        scratch_types=[
            pltpu.VMEM((c,), jnp.int32),
            pltpu.VMEM((c,), jnp.int32),
            pltpu.VMEM((c, h), jnp.float32),
            pltpu.VMEM((c, h), jnp.float32),
            pltpu.VMEM((c, h), jnp.float32),
            pltpu.VMEM((h,), jnp.float32),
            pltpu.VMEM((c, 16), jnp.float32),
            pltpu.SemaphoreType.DMA,
        ],
    )
    def k(src_hbm, dst_hbm, ha_hbm, hb_hbm, hew_hbm, w2_hbm, out_hbm, srcv,
          dstv, rowsa, rowsb, rowse, w2v, outv, sem):
        cid = lax.axis_index("c")
        sid = lax.axis_index("s")
        wid = cid * NS + sid
        pltpu.sync_copy(w2_hbm, w2v)
        w2r = [w2v[pl.ds(kk * 16, 16)] for kk in range(kh)]
        ebase = wid * ept

        def chunk_body(cc, _):
            off = ebase + cc * c
            pltpu.sync_copy(src_hbm.at[pl.ds(off, c)], srcv)
            pltpu.sync_copy(dst_hbm.at[pl.ds(off, c)], dstv)
            pltpu.async_copy(ha_hbm.at[srcv], rowsa, sem).wait()
            pltpu.async_copy(hb_hbm.at[dstv], rowsb, sem).wait()
            pltpu.sync_copy(hew_hbm.at[pl.ds(off, c)], rowse)

            @plsc.parallel_loop(0, c, unroll=4)
            def _(ee):
                acc = jnp.zeros((16,), jnp.float32)
                for kk in range(kh):
                    q = (rowsa[ee, pl.ds(kk * 16, 16)]
                         + rowsb[ee, pl.ds(kk * 16, 16)]
                         + rowse[ee, pl.ds(kk * 16, 16)])
                    q = jnp.maximum(q, 0.0)
                    acc = acc + q * w2r[kk]
                outv[ee, :] = acc
            pltpu.sync_copy(outv, out_hbm.at[pl.ds(off, c)])
            return 0

        lax.fori_loop(0, nchunk, chunk_body, 0)

    return k(src, dst, ha, hb, hew, w2)


def kernel(x, edge_index, edge_attr, enc_node_W, enc_node_b, enc_edge_W,
           enc_edge_b, gat_W, gat_a_src, gat_a_dst, gat_b, dec_W1, dec_b1,
           dec_W2, dec_b2):
    n, dn = x.shape
    e = edge_index.shape[1]
    hh = enc_node_W.shape[1]
    ll = gat_W.shape[0]

    bm_n = 1000 if n % 1000 == 0 else n
    bm_e = 2000 if e % 2000 == 0 else e
    c = 80 if (e // NW) % 80 == 0 else 16

    src = edge_index[0]
    dst = edge_index[1]

    h = _mm_bias_relu(x, enc_node_W, enc_node_b.reshape(1, hh), bm_n)

    for l in range(ll):
        a2 = jnp.stack([gat_a_src[l], gat_a_dst[l]], axis=1)
        hw, ssc, sdc = _gat_dense(h, gat_W[l], a2, bm_n)
        pn, pd = _gat_edge_sc(src, dst, ssc[:, 0], sdc[:, 0], hw,
                              n=n, e=e, h=hh, c=c)
        dcol = pd.reshape(NC, -1, 1)[:, :n, :]
        h = _combine(pn, dcol, gat_b[l].reshape(1, hh), bm_n)

    w1a = dec_W1[:hh]
    w1b = dec_W1[hh:2 * hh]
    w1c = dec_W1[2 * hh:]
    ha, hb = _dec_tables(h, w1a, w1b, bm_n)
    hew = _edge_mlp(edge_attr, enc_edge_W, enc_edge_b.reshape(1, hh), w1c,
                    dec_b1.reshape(1, hh), bm_e)
    part = _dec_edge_sc(src, dst, ha, hb, hew, dec_W2[:, 0],
                        n=n, e=e, h=hh, c=c)
    out = _dec_reduce(part, dec_b2.reshape(1, 1), bm_e)
    return out


# speedup vs baseline: 13.2035x; 1.2688x over previous
import functools

import jax
import jax.numpy as jnp
from jax import lax
from jax.experimental import pallas as pl
from jax.experimental.pallas import tpu as pltpu
from jax.experimental.pallas import tpu_sc as plsc

NC = 2
NS = 16
NW = NC * NS


def _mm_bias_relu_body(x_ref, w_ref, b_ref, o_ref):
    y = jnp.dot(x_ref[...], w_ref[...], preferred_element_type=jnp.float32)
    o_ref[...] = jnp.maximum(y + b_ref[...], 0.0)


def _mm_bias_relu(x, w, b2d, bm):
    m, k = x.shape
    n = w.shape[1]
    return pl.pallas_call(
        _mm_bias_relu_body,
        grid=(m // bm,),
        in_specs=[
            pl.BlockSpec((bm, k), lambda i: (i, 0)),
            pl.BlockSpec((k, n), lambda i: (0, 0)),
            pl.BlockSpec((1, n), lambda i: (0, 0)),
        ],
        out_specs=pl.BlockSpec((bm, n), lambda i: (i, 0)),
        out_shape=jax.ShapeDtypeStruct((m, n), jnp.float32),
    )(x, w, b2d)


def _gat_dense_body(h_ref, w_ref, a_ref, hw_ref, ss_ref, sd_ref):
    hw = jnp.dot(h_ref[...], w_ref[...], preferred_element_type=jnp.float32)
    hw_ref[...] = hw
    sv = jnp.dot(hw, a_ref[...], preferred_element_type=jnp.float32)
    ss_ref[...] = sv[:, 0:1]
    sd_ref[...] = sv[:, 1:2]


def _gat_dense(h, w, a2, bm):
    m, k = h.shape
    n = w.shape[1]
    return pl.pallas_call(
        _gat_dense_body,
        grid=(m // bm,),
        in_specs=[
            pl.BlockSpec((bm, k), lambda i: (i, 0)),
            pl.BlockSpec((k, n), lambda i: (0, 0)),
            pl.BlockSpec((n, 2), lambda i: (0, 0)),
        ],
        out_specs=[
            pl.BlockSpec((bm, n), lambda i: (i, 0)),
            pl.BlockSpec((bm, 1), lambda i: (i, 0)),
            pl.BlockSpec((bm, 1), lambda i: (i, 0)),
        ],
        out_shape=[
            jax.ShapeDtypeStruct((m, n), jnp.float32),
            jax.ShapeDtypeStruct((m, 1), jnp.float32),
            jax.ShapeDtypeStruct((m, 1), jnp.float32),
        ],
    )(h, w, a2)


def _combine_body(pn_ref, pd_ref, b_ref, o_ref):
    num = pn_ref[0] + pn_ref[1]
    den = pd_ref[0] + pd_ref[1]
    o_ref[...] = jnp.maximum(num / (den + 1e-16) + b_ref[...], 0.0)


def _combine(pn, dcol, b2d, bm):
    _, m, n = pn.shape
    return pl.pallas_call(
        _combine_body,
        grid=(m // bm,),
        in_specs=[
            pl.BlockSpec((2, bm, n), lambda i: (0, i, 0)),
            pl.BlockSpec((2, bm, 1), lambda i: (0, i, 0)),
            pl.BlockSpec((1, n), lambda i: (0, 0)),
        ],
        out_specs=pl.BlockSpec((bm, n), lambda i: (i, 0)),
        out_shape=jax.ShapeDtypeStruct((m, n), jnp.float32),
    )(pn, dcol, b2d)


def _dec_tables_body(h_ref, wa_ref, wb_ref, a_ref, b_ref):
    a_ref[...] = jnp.dot(h_ref[...], wa_ref[...], preferred_element_type=jnp.float32)
    b_ref[...] = jnp.dot(h_ref[...], wb_ref[...], preferred_element_type=jnp.float32)


def _dec_tables(h, wa, wb, bm):
    m, k = h.shape
    n = wa.shape[1]
    return pl.pallas_call(
        _dec_tables_body,
        grid=(m // bm,),
        in_specs=[
            pl.BlockSpec((bm, k), lambda i: (i, 0)),
            pl.BlockSpec((k, n), lambda i: (0, 0)),
            pl.BlockSpec((k, n), lambda i: (0, 0)),
        ],
        out_specs=[
            pl.BlockSpec((bm, n), lambda i: (i, 0)),
            pl.BlockSpec((bm, n), lambda i: (i, 0)),
        ],
        out_shape=[
            jax.ShapeDtypeStruct((m, n), jnp.float32),
            jax.ShapeDtypeStruct((m, n), jnp.float32),
        ],
    )(h, wa, wb)


def _edge_mlp_body(ea_ref, we_ref, be_ref, w1c_ref, b1_ref, o_ref):
    he = jnp.dot(ea_ref[...], we_ref[...], preferred_element_type=jnp.float32)
    he = jnp.maximum(he + be_ref[...], 0.0)
    o_ref[...] = jnp.dot(he, w1c_ref[...], preferred_element_type=jnp.float32) + b1_ref[...]


def _edge_mlp(ea, we, be2d, w1c, b12d, bm):
    m, k = ea.shape
    n = we.shape[1]
    return pl.pallas_call(
        _edge_mlp_body,
        grid=(m // bm,),
        in_specs=[
            pl.BlockSpec((bm, k), lambda i: (i, 0)),
            pl.BlockSpec((k, n), lambda i: (0, 0)),
            pl.BlockSpec((1, n), lambda i: (0, 0)),
            pl.BlockSpec((n, n), lambda i: (0, 0)),
            pl.BlockSpec((1, n), lambda i: (0, 0)),
        ],
        out_specs=pl.BlockSpec((bm, n), lambda i: (i, 0)),
        out_shape=jax.ShapeDtypeStruct((m, n), jnp.float32),
    )(ea, we, be2d, w1c, b12d)


def _dec_reduce_body(p_ref, b2_ref, o_ref):
    o_ref[...] = jnp.sum(p_ref[...], axis=1, keepdims=True) + b2_ref[...]


def _dec_reduce(p, b2d, bm):
    m = p.shape[0]
    return pl.pallas_call(
        _dec_reduce_body,
        grid=(m // bm,),
        in_specs=[
            pl.BlockSpec((bm, 16), lambda i: (i, 0)),
            pl.BlockSpec((1, 1), lambda i: (0, 0)),
        ],
        out_specs=pl.BlockSpec((bm, 1), lambda i: (i, 0)),
        out_shape=jax.ShapeDtypeStruct((m, 1), jnp.float32),
    )(p, b2d)


def _gat_edge_sc(src, dst, ss, sd, hw, *, n, e, h, c):
    ept = e // NW
    nchunk = ept // c
    kh = h // 16
    nrc = (n // 16 + NS - 1) // NS
    nd = ((n // 128) + NS - 1) // NS * NS
    ndc = (nd // 16 + NS - 1) // NS
    maxcd = nd // 16 - 1

    mesh = plsc.VectorSubcoreMesh(core_axis_name="c", subcore_axis_name="s",
                                  num_cores=NC, num_subcores=NS)

    @functools.partial(
        pl.kernel,
        out_type=(
            jax.ShapeDtypeStruct((NC * n, h), jnp.float32),
            jax.ShapeDtypeStruct((NC * nd, 128), jnp.float32),
        ),
        mesh=mesh,
        compiler_params=pltpu.CompilerParams(needs_layout_passes=False),
        scratch_types=[
            pltpu.VMEM((c,), jnp.int32),
            pltpu.VMEM((c,), jnp.int32),
            pltpu.VMEM((c,), jnp.int32),
            pltpu.VMEM((c, h), jnp.float32),
            pltpu.VMEM((c,), jnp.float32),
            pltpu.VMEM((c,), jnp.float32),
            pltpu.VMEM((c,), jnp.float32),
            pltpu.VMEM((c, 128), jnp.float32),
            pltpu.VMEM((16,), jnp.int32),
            pltpu.VMEM((16, h), jnp.float32),
            pltpu.VMEM((16, 128), jnp.float32),
            pltpu.VMEM_SHARED((n, h), jnp.float32),
            pltpu.VMEM_SHARED((nd, 128), jnp.float32),
            pltpu.SemaphoreType.DMA,
        ],
    )
    def k(src_hbm, dst_hbm, ss_hbm, sd_hbm, hw_hbm, pn_hbm, pd_hbm,
          srcv, dstv, d8v, rows, ssg, sdg, wv, wst, ridx, bn, bd,
          accn, accd, sem):
        cid = lax.axis_index("c")
        sid = lax.axis_index("s")
        wid = cid * NS + sid

        lane = lax.iota(jnp.int32, 16)
        zv = jnp.zeros((16,), jnp.float32)
        for r in range(16):
            for kk in range(kh):
                bn[r, pl.ds(kk * 16, 16)] = zv
        for r in range(c):
            for kk in range(8):
                wst[r, pl.ds(kk * 16, 16)] = zv

        maxc = n // 16 - 1

        def zchunk(g, _):
            ch = jnp.minimum(g * NS + sid, maxc)
            ridx[:] = ch * 16 + lane
            pltpu.sync_copy(bn, accn.at[ridx])
            return 0

        lax.fori_loop(0, nrc, zchunk, 0)

        def zchunkd(g, _):
            ch = jnp.minimum(g * NS + sid, maxcd)
            ridx[:] = ch * 16 + lane
            pltpu.sync_copy(bn, accd.at[ridx])
            return 0

        lax.fori_loop(0, ndc, zchunkd, 0)
        plsc.subcore_barrier()

        ebase = wid * ept

        def chunk_body(cc, _):
            off = ebase + cc * c
            pltpu.sync_copy(src_hbm.at[pl.ds(off, c)], srcv)
            pltpu.sync_copy(dst_hbm.at[pl.ds(off, c)], dstv)
            cp1 = pltpu.async_copy(hw_hbm.at[srcv], rows, sem)
            cp2 = pltpu.async_copy(ss_hbm.at[srcv], ssg, sem)
            cp3 = pltpu.async_copy(sd_hbm.at[dstv], sdg, sem)
            cp1.wait()
            cp2.wait()
            cp3.wait()
            for g in range(c // 16):
                s = ssg[pl.ds(g * 16, 16)] + sdg[pl.ds(g * 16, 16)]
                s = jnp.where(s > 0, s, 0.2 * s)
                w = jnp.exp(s)
                wv[pl.ds(g * 16, 16)] = w
                di = dstv[pl.ds(g * 16, 16)]
                d8v[pl.ds(g * 16, 16)] = lax.shift_right_logical(di, 7)
                plsc.store_scatter(wst, [g * 16 + lane, di & 127], w)

            @plsc.parallel_loop(0, c, unroll=4)
            def _(ee):
                wsp = plsc.load_gather(wv, [jnp.full((16,), ee, jnp.int32)])
                for kk in range(kh):
                    rows[ee, pl.ds(kk * 16, 16)] = rows[ee, pl.ds(kk * 16, 16)] * wsp
            pltpu.sync_copy(rows, accn.at[dstv], add=True)
            pltpu.sync_copy(wst, accd.at[d8v], add=True)
            for g in range(c // 16):
                di = dstv[pl.ds(g * 16, 16)]
                plsc.store_scatter(wst, [g * 16 + lane, di & 127], zv)
            return 0

        lax.fori_loop(0, nchunk, chunk_body, 0)
        plsc.subcore_barrier()

        def cchunk(g, _):
            ch = jnp.minimum(g * NS + sid, maxc)
            ridx[:] = ch * 16 + lane
            pltpu.async_copy(accn.at[ridx], bn, sem).wait()
            r = pl.multiple_of(cid * n, 8) + pl.multiple_of(ch * 16, 8)
            pltpu.sync_copy(bn, pn_hbm.at[pl.ds(r, 16)])
            return 0

        lax.fori_loop(0, nrc, cchunk, 0)

        def cchunkd(g, _):
            ch = jnp.minimum(g * NS + sid, maxcd)
            ridx[:] = ch * 16 + lane
            pltpu.async_copy(accd.at[ridx], bd, sem).wait()
            r = pl.multiple_of(cid * nd, 8) + pl.multiple_of(ch * 16, 8)
            pltpu.sync_copy(bd, pd_hbm.at[pl.ds(r, 16)])
            return 0

        lax.fori_loop(0, ndc, cchunkd, 0)

    pn2, pd2 = k(src, dst, ss, sd, hw)
    return pn2.reshape(NC, n, h), pd2.reshape(NC, nd, 128)


def _dec_edge_sc(src, dst, ha, hb, hew, w2, *, n, e, h, c):
    ept = e // NW
    nchunk = ept // c
    npair = (nchunk - 1) // 2
    kh = h // 16

    mesh = plsc.VectorSubcoreMesh(core_axis_name="c", subcore_axis_name="s",
                                  num_cores=NC, num_subcores=NS)

    @functools.partial(
        pl.kernel,
        out_type=jax.ShapeDtypeStruct((e, 16), jnp.float32),
        mesh=mesh,
        compiler_params=pltpu.CompilerParams(needs_layout_passes=False),
        scratch_types=[
            pltpu.VMEM((2, c), jnp.int32),
            pltpu.VMEM((2, c), jnp.int32),
            pltpu.VMEM((2, c, h), jnp.float32),
            pltpu.VMEM((2, c, h), jnp.float32),
            pltpu.VMEM((2, c, h), jnp.float32),
            pltpu.VMEM((h,), jnp.float32),
            pltpu.VMEM((c, 16), jnp.float32),
            pltpu.SemaphoreType.DMA,
            pltpu.SemaphoreType.DMA,
            pltpu.SemaphoreType.DMA,
            pltpu.SemaphoreType.DMA,
        ],
    )
    def k(src_hbm, dst_hbm, ha_hbm, hb_hbm, hew_hbm, w2_hbm, out_hbm, srcv2,
          dstv2, rowsa2, rowsb2, rowse2, w2v, outv, semi0, semi1, semg0, semg1):
        cid = lax.axis_index("c")
        sid = lax.axis_index("s")
        wid = cid * NS + sid
        pltpu.sync_copy(w2_hbm, w2v)
        w2r = [w2v[pl.ds(kk * 16, 16)] for kk in range(kh)]
        ebase = wid * ept
        semi = [semi0, semi1]
        semg = [semg0, semg1]
        srcv = [srcv2.at[0], srcv2.at[1]]
        dstv = [dstv2.at[0], dstv2.at[1]]
        rowsa = [rowsa2.at[0], rowsa2.at[1]]
        rowsb = [rowsb2.at[0], rowsb2.at[1]]
        rowse = [rowse2.at[0], rowse2.at[1]]

        def issue_idx(ch, p):
            off = ebase + ch * c
            pltpu.async_copy(src_hbm.at[pl.ds(off, c)], srcv[p], semi[p])
            pltpu.async_copy(dst_hbm.at[pl.ds(off, c)], dstv[p], semi[p])

        def wait_idx(p):
            pltpu.make_async_copy(src_hbm.at[pl.ds(0, c)], srcv[p], semi[p]).wait()
            pltpu.make_async_copy(dst_hbm.at[pl.ds(0, c)], dstv[p], semi[p]).wait()

        def issue_gat(ch, p):
            off = ebase + ch * c
            pltpu.async_copy(ha_hbm.at[srcv[p]], rowsa[p], semg[p])
            pltpu.async_copy(hb_hbm.at[dstv[p]], rowsb[p], semg[p])
            pltpu.async_copy(hew_hbm.at[pl.ds(off, c)], rowse[p], semg[p])

        def wait_gat(p):
            pltpu.make_async_copy(ha_hbm.at[srcv[p]], rowsa[p], semg[p]).wait()
            pltpu.make_async_copy(hb_hbm.at[dstv[p]], rowsb[p], semg[p]).wait()
            pltpu.make_async_copy(hew_hbm.at[pl.ds(0, c)], rowse[p], semg[p]).wait()

        def compute(ch, p):
            ra, rb, re = rowsa[p], rowsb[p], rowse[p]

            @plsc.parallel_loop(0, c, unroll=4)
            def _(ee):
                acc = jnp.zeros((16,), jnp.float32)
                for kk in range(kh):
                    q = (ra[ee, pl.ds(kk * 16, 16)]
                         + rb[ee, pl.ds(kk * 16, 16)]
                         + re[ee, pl.ds(kk * 16, 16)])
                    q = jnp.maximum(q, 0.0)
                    acc = acc + q * w2r[kk]
                outv[ee, :] = acc

            off = ebase + ch * c
            pltpu.sync_copy(outv, out_hbm.at[pl.ds(off, c)])

        issue_idx(0, 0)
        issue_idx(1, 1)
        wait_idx(0)
        issue_gat(0, 0)

        def pair_body(g, _):
            for ph in range(2):
                cc = 2 * g + ph
                q = 1 - ph
                wait_gat(ph)
                wait_idx(q)
                issue_gat(cc + 1, q)
                compute(cc, ph)
                issue_idx(jnp.minimum(cc + 2, nchunk - 1), ph)
            return 0

        lax.fori_loop(0, npair, pair_body, 0)
        wait_gat(0)
        compute(nchunk - 1, 0)
        wait_idx(1)

    return k(src, dst, ha, hb, hew, w2)


def kernel(x, edge_index, edge_attr, enc_node_W, enc_node_b, enc_edge_W,
           enc_edge_b, gat_W, gat_a_src, gat_a_dst, gat_b, dec_W1, dec_b1,
           dec_W2, dec_b2):
    n, dn = x.shape
    e = edge_index.shape[1]
    hh = enc_node_W.shape[1]
    ll = gat_W.shape[0]

    bm_n = 1000 if n % 1000 == 0 else n
    bm_e = 2000 if e % 2000 == 0 else e
    c = 80 if (e // NW) % 80 == 0 else 16

    src = edge_index[0]
    dst = edge_index[1]

    h = _mm_bias_relu(x, enc_node_W, enc_node_b.reshape(1, hh), bm_n)

    for l in range(ll):
        a2 = jnp.stack([gat_a_src[l], gat_a_dst[l]], axis=1)
        hw, ssc, sdc = _gat_dense(h, gat_W[l], a2, bm_n)
        pn, pd = _gat_edge_sc(src, dst, ssc[:, 0], sdc[:, 0], hw,
                              n=n, e=e, h=hh, c=c)
        dcol = pd.reshape(NC, -1, 1)[:, :n, :]
        h = _combine(pn, dcol, gat_b[l].reshape(1, hh), bm_n)

    w1a = dec_W1[:hh]
    w1b = dec_W1[hh:2 * hh]
    w1c = dec_W1[2 * hh:]
    ha, hb = _dec_tables(h, w1a, w1b, bm_n)
    hew = _edge_mlp(edge_attr, enc_edge_W, enc_edge_b.reshape(1, hh), w1c,
                    dec_b1.reshape(1, hh), bm_e)
    part = _dec_edge_sc(src, dst, ha, hb, hew, dec_W2[:, 0],
                        n=n, e=e, h=hh, c=c)
    out = _dec_reduce(part, dec_b2.reshape(1, 1), bm_e)
    return out
